# baseline (speedup 1.0000x reference)
import math

import jax
import jax.numpy as jnp
from jax.experimental import pallas as pl
from jax.experimental.pallas import tpu as pltpu

_MAX_REL = 128
_SQ = 2048
_SK = 2048
_TQ = 256
_TK = 512
_L = ((_TQ + _TK - 1 + 127) // 128) * 128
_G = math.gcd(_TQ, _TK)
_ND = (_SQ - _TQ + _SK - _TK) // _G + 1
_HEAD = _SQ - _MAX_REL - 1


def _add_bias_body(f_ref, x_ref, o_ref):
    f = f_ref[0, 0, :]
    fb = jnp.broadcast_to(f[None, :], (_TQ, _L))
    bias = pltpu.roll(fb, _L - _TQ + 1, axis=1, stride=1, stride_axis=0)
    o_ref[...] = x_ref[...] + bias[None, :, :_TK]


def _run(x, f_all):
    b = x.shape[0]
    grid = (_SQ // _TQ, _SK // _TK)

    def f_idx(qi, ki):
        return ((ki * _TK - qi * _TQ + _SQ - _TQ) // _G, 0, 0)

    return pl.pallas_call(
        _add_bias_body,
        grid=grid,
        in_specs=[
            pl.BlockSpec((1, 1, _L), f_idx),
            pl.BlockSpec((b, _TQ, _TK), lambda qi, ki: (0, qi, ki)),
        ],
        out_specs=pl.BlockSpec((b, _TQ, _TK), lambda qi, ki: (0, qi, ki)),
        out_shape=jax.ShapeDtypeStruct(x.shape, x.dtype),
    )(f_all, x)


def kernel(inputs, relative_biases):
    t = relative_biases
    tail = _G * (_ND - 1) + _L - _HEAD - (2 * _MAX_REL + 1)
    e = jnp.concatenate(
        [jnp.full((_HEAD,), t[0], t.dtype), t, jnp.full((tail,), t[256], t.dtype)]
    )
    f_all = jnp.stack([e[_G * d : _G * d + _L] for d in range(_ND)])
    f_all = f_all.reshape(_ND, 1, _L)

    out_a = _run(inputs[:4], f_all)
    out_b = _run(inputs[4:], f_all)
    return jnp.concatenate([out_a, out_b], axis=0)


# speedup vs baseline: 2.6940x; 2.6940x over previous
import dataclasses

import jax
import jax.numpy as jnp
from jax import lax
from jax.experimental import pallas as pl
from jax.experimental.pallas import tpu as pltpu
from jax.experimental.pallas import tpu_sc as plsc

_MAX_REL = 128
_SQ = 2048
_TQ = 256
_TK = 256
_L = 512
_ND = 15
_SHIFT = _SQ - _MAX_REL - 1


def _sc_windows(t_hbm, f_hbm, t_v, row_v):
    wid = lax.axis_index("s") * 2 + lax.axis_index("c")

    @pl.when(wid < _ND)
    def _():
        pltpu.sync_copy(t_hbm, t_v)
        lane = lax.iota(jnp.int32, 16)
        for c in range(_L // 16):
            idx = jnp.clip(256 * wid + 16 * c + lane - _SHIFT, 0, 2 * _MAX_REL)
            row_v[pl.ds(16 * c, 16)] = plsc.load_gather(t_v, [idx])
        pltpu.sync_copy(row_v, f_hbm.at[wid])


def _build_windows(relative_biases):
    t_pad = jnp.concatenate(
        [relative_biases, jnp.full((255,), relative_biases[256], relative_biases.dtype)]
    )
    mesh = plsc.VectorSubcoreMesh(core_axis_name="c", subcore_axis_name="s")
    cp = pltpu.CompilerParams()
    if "needs_layout_passes" in pltpu.CompilerParams.__dataclass_fields__:
        cp = dataclasses.replace(cp, needs_layout_passes=False)
    f_all = pl.kernel(
        _sc_windows,
        mesh=mesh,
        compiler_params=cp,
        out_type=jax.ShapeDtypeStruct((_ND, _L), jnp.float32),
        scratch_types=[
            pltpu.VMEM((512,), jnp.float32),
            pltpu.VMEM((_L,), jnp.float32),
        ],
    )(t_pad)
    return f_all.reshape(_ND, 1, _L)


def _add_bias_body(f_ref, x_ref, o_ref):
    f = f_ref[0, 0, :]
    fb = jnp.broadcast_to(f[None, :], (_TQ, _L))
    bias = pltpu.roll(fb, _L - _TQ + 1, axis=1, stride=1, stride_axis=0)
    o_ref[...] = x_ref[...] + bias[None, :, :_TK]


def kernel(inputs, relative_biases):
    f_all = _build_windows(relative_biases)

    b = inputs.shape[0]
    grid = (_SQ // _TQ, _SQ // _TK)

    def f_idx(qi, ki):
        return ((ki * _TK - qi * _TQ + _SQ - _TQ) // 256, 0, 0)

    return pl.pallas_call(
        _add_bias_body,
        grid=grid,
        in_specs=[
            pl.BlockSpec((1, 1, _L), f_idx),
            pl.BlockSpec((b, _TQ, _TK), lambda qi, ki: (0, qi, ki)),
        ],
        out_specs=pl.BlockSpec((b, _TQ, _TK), lambda qi, ki: (0, qi, ki)),
        out_shape=jax.ShapeDtypeStruct(inputs.shape, inputs.dtype),
    )(f_all, inputs)
